# baseline (speedup 1.0000x reference)
import functools

import jax
import jax.numpy as jnp
import numpy as np
from jax import lax
from jax.experimental import pallas as pl
from jax.experimental.pallas import tpu as pltpu
from jax.experimental.pallas import tpu_sc as plsc

POOL = 20
TOPK = 5
NTOK = 197
DIM = 768
NLAYERS = 12
COLS = NTOK + 2 * POOL
KPAD = 128
STRIPE = 2 * POOL
TILE_WORDS = NTOK * COLS
TILE_PAD = 46704
NSTRIPE = NTOK * STRIPE
NSTRIPE_PAD = 7888
NCHUNK = NSTRIPE_PAD // 16


def _tc_body(x_ref, kp_ref, stripe_ref, dacc_ref):
    b = pl.program_id(0)

    xb = x_ref[0]
    xn = xb / jnp.maximum(jnp.sqrt(jnp.sum(xb * xb, axis=1, keepdims=True)), 1e-12)
    kp = kp_ref[...]
    kn = kp / jnp.maximum(jnp.sqrt(jnp.sum(kp * kp, axis=1, keepdims=True)), 1e-12)
    sim = lax.dot_general(
        xn, kn, (((1,), (1,)), ((), ())), preferred_element_type=jnp.float32
    )

    r = lax.broadcasted_iota(jnp.int32, (NTOK, KPAD), 0)
    c = lax.broadcasted_iota(jnp.int32, (NTOK, KPAD), 1)
    valid = ((r == 0) & (c < POOL)) | ((r != 0) & (c >= POOL) & (c < 2 * POOL))
    simv = jnp.where(valid, sim, -2.0)

    ind = jnp.zeros((NTOK, KPAD), jnp.float32)
    ssum = jnp.zeros((NTOK, 1), jnp.float32)
    for _ in range(TOPK):
        m = jnp.max(simv, axis=1, keepdims=True)
        first = jnp.min(jnp.where(simv == m, c, KPAD), axis=1, keepdims=True)
        onehot = c == first
        ind = jnp.where(onehot, 1.0, ind)
        ssum = ssum + m
        simv = jnp.where(onehot, -3.0, simv)

    stripe_ref[0] = ind[:, :STRIPE]

    rr = lax.broadcasted_iota(jnp.int32, (NTOK, 1), 0)
    cls_sum = jnp.sum(jnp.where(rr == 0, ssum, 0.0))
    img_sum = jnp.sum(jnp.where(rr == 0, 0.0, ssum))
    ar = lax.broadcasted_iota(jnp.int32, (8, 128), 0)
    ac = lax.broadcasted_iota(jnp.int32, (8, 128), 1)
    part = jnp.where((ar == 0) & (ac == 0), cls_sum, 0.0) + jnp.where(
        (ar == 0) & (ac == 1), img_sum, 0.0
    )

    @pl.when(b == 0)
    def _():
        dacc_ref[...] = jnp.zeros((8, 128), jnp.float32)

    dacc_ref[...] += part


def _tc_select(x, kp):
    B = x.shape[0]
    return pl.pallas_call(
        _tc_body,
        grid=(B,),
        in_specs=[
            pl.BlockSpec((1, NTOK, DIM), lambda b: (b, 0, 0)),
            pl.BlockSpec((KPAD, DIM), lambda b: (0, 0)),
        ],
        out_specs=[
            pl.BlockSpec((1, NTOK, STRIPE), lambda b: (b, 0, 0)),
            pl.BlockSpec((8, 128), lambda b: (0, 0)),
        ],
        out_shape=[
            jax.ShapeDtypeStruct((B, NTOK, STRIPE), jnp.float32),
            jax.ShapeDtypeStruct((8, 128), jnp.float32),
        ],
    )(x, kp)


def _make_sc_fill(B):
    info = plsc.get_sparse_core_info()
    NC, NS = info.num_cores, info.num_subcores
    NW = NC * NS
    BPW = B // NW
    mesh = plsc.VectorSubcoreMesh(core_axis_name="c", subcore_axis_name="s")

    @functools.partial(
        pl.kernel,
        mesh=mesh,
        compiler_params=pltpu.CompilerParams(
            use_tc_tiling_on_sc=False, needs_layout_passes=False
        ),
        out_type=jax.ShapeDtypeStruct((B, NLAYERS, TILE_WORDS), jnp.float32),
        scratch_types=[
            pltpu.VMEM((TILE_PAD,), jnp.float32),
            pltpu.VMEM((NSTRIPE_PAD,), jnp.float32),
            pltpu.VMEM((NSTRIPE_PAD,), jnp.int32),
            pltpu.SemaphoreType.DMA,
        ],
    )
    def sc_fill(
        stripes_hbm,
        ones_hbm,
        wi_hbm,
        out_hbm,
        tile_v,
        stripe_v,
        wi_v,
        sem,
    ):
        wid = lax.axis_index("s") * NC + lax.axis_index("c")
        pltpu.sync_copy(ones_hbm, tile_v)
        pltpu.sync_copy(wi_hbm, wi_v)
        for bb in range(BPW):
            b = wid * BPW + bb
            pltpu.sync_copy(stripes_hbm.at[b], stripe_v.at[pl.ds(0, NSTRIPE)])

            def _chunk(k, carry):
                wk = wi_v[pl.ds(k * 16, 16)]
                v = stripe_v[pl.ds(k * 16, 16)]
                plsc.store_scatter(tile_v, [wk], v)
                return carry

            lax.fori_loop(0, NCHUNK, _chunk, 0)
            copies = [
                pltpu.async_copy(
                    tile_v.at[pl.ds(0, TILE_WORDS)], out_hbm.at[b, l], sem
                )
                for l in range(NLAYERS)
            ]
            for cp in copies:
                cp.wait()

    return sc_fill


def kernel(x, keys_cls, keys_image, layer):
    B = x.shape[0]
    kc = jnp.take(keys_cls, layer, axis=0)
    ki = jnp.take(keys_image, layer, axis=0)
    kp = (
        jnp.zeros((KPAD, DIM), jnp.float32)
        .at[:POOL]
        .set(kc)
        .at[POOL : 2 * POOL]
        .set(ki)
    )

    stripes, dacc = _tc_select(x, kp)
    stripes_flat = stripes.reshape(B, NSTRIPE)

    j = np.arange(NSTRIPE_PAD)
    wj = np.where(
        j < NSTRIPE,
        COLS * (j // STRIPE) + 1 + (j % STRIPE),
        TILE_WORDS + (j - NSTRIPE),
    )
    wi = jnp.asarray(wj, jnp.int32)
    ones = jnp.ones((TILE_PAD,), jnp.float32)

    flat = _make_sc_fill(B)(stripes_flat, ones, wi)
    mask = flat.reshape(B, NLAYERS, NTOK, COLS)

    dist = (1.0 - dacc[0, 0] / (B * TOPK)) + (
        1.0 - dacc[0, 1] / (B * (NTOK - 1) * TOPK)
    )
    return (mask, dist)


# speedup vs baseline: 1.3833x; 1.3833x over previous
import functools

import jax
import jax.numpy as jnp
import numpy as np
from jax import lax
from jax.experimental import pallas as pl
from jax.experimental.pallas import tpu as pltpu
from jax.experimental.pallas import tpu_sc as plsc

POOL = 20
TOPK = 5
NTOK = 197
DIM = 768
NLAYERS = 12
COLS = NTOK + 2 * POOL
KPAD = 128
STRIPE = 2 * POOL
TILE_WORDS = NTOK * COLS
TILE_PAD = 46704
NSTRIPE = NTOK * STRIPE
NSTRIPE_PAD = 7888
NCHUNK = NSTRIPE_PAD // 16


def _tc_body(x_ref, kp_ref, stripe_ref, dacc_ref):
    b = pl.program_id(0)

    xb = x_ref[0]
    xn = xb / jnp.maximum(jnp.sqrt(jnp.sum(xb * xb, axis=1, keepdims=True)), 1e-12)
    kp = kp_ref[...]
    kn = kp / jnp.maximum(jnp.sqrt(jnp.sum(kp * kp, axis=1, keepdims=True)), 1e-12)
    sim = lax.dot_general(
        xn, kn, (((1,), (1,)), ((), ())), preferred_element_type=jnp.float32
    )

    r = lax.broadcasted_iota(jnp.int32, (NTOK, KPAD), 0)
    c = lax.broadcasted_iota(jnp.int32, (NTOK, KPAD), 1)
    valid = ((r == 0) & (c < POOL)) | ((r != 0) & (c >= POOL) & (c < 2 * POOL))
    simv = jnp.where(valid, sim, -2.0)

    ind = jnp.zeros((NTOK, KPAD), jnp.float32)
    ssum = jnp.zeros((NTOK, 1), jnp.float32)
    for _ in range(TOPK):
        m = jnp.max(simv, axis=1, keepdims=True)
        first = jnp.min(jnp.where(simv == m, c, KPAD), axis=1, keepdims=True)
        onehot = c == first
        ind = jnp.where(onehot, 1.0, ind)
        ssum = ssum + m
        simv = jnp.where(onehot, -3.0, simv)

    stripe_ref[0] = ind[:, :STRIPE]

    rr = lax.broadcasted_iota(jnp.int32, (NTOK, 1), 0)
    cls_sum = jnp.sum(jnp.where(rr == 0, ssum, 0.0))
    img_sum = jnp.sum(jnp.where(rr == 0, 0.0, ssum))
    ar = lax.broadcasted_iota(jnp.int32, (8, 128), 0)
    ac = lax.broadcasted_iota(jnp.int32, (8, 128), 1)
    part = jnp.where((ar == 0) & (ac == 0), cls_sum, 0.0) + jnp.where(
        (ar == 0) & (ac == 1), img_sum, 0.0
    )

    @pl.when(b == 0)
    def _():
        dacc_ref[...] = jnp.zeros((8, 128), jnp.float32)

    dacc_ref[...] += part


def _tc_select(x, kp):
    B = x.shape[0]
    return pl.pallas_call(
        _tc_body,
        grid=(B,),
        in_specs=[
            pl.BlockSpec((1, NTOK, DIM), lambda b: (b, 0, 0)),
            pl.BlockSpec((KPAD, DIM), lambda b: (0, 0)),
        ],
        out_specs=[
            pl.BlockSpec((1, NTOK, STRIPE), lambda b: (b, 0, 0)),
            pl.BlockSpec((8, 128), lambda b: (0, 0)),
        ],
        out_shape=[
            jax.ShapeDtypeStruct((B, NTOK, STRIPE), jnp.float32),
            jax.ShapeDtypeStruct((8, 128), jnp.float32),
        ],
    )(x, kp)


def _make_sc_fill(B):
    info = plsc.get_sparse_core_info()
    NC, NS = info.num_cores, info.num_subcores
    NW = NC * NS
    BPW = B // NW
    mesh = plsc.VectorSubcoreMesh(core_axis_name="c", subcore_axis_name="s")

    @functools.partial(
        pl.kernel,
        mesh=mesh,
        compiler_params=pltpu.CompilerParams(
            use_tc_tiling_on_sc=False, needs_layout_passes=False
        ),
        out_type=jax.ShapeDtypeStruct((B, NLAYERS, NTOK, COLS), jnp.float32),
        scratch_types=[
            pltpu.VMEM((NTOK, COLS), jnp.float32),
            pltpu.VMEM((NSTRIPE_PAD,), jnp.float32),
            pltpu.VMEM((NSTRIPE_PAD,), jnp.int32),
            pltpu.VMEM((NSTRIPE_PAD,), jnp.int32),
            pltpu.SemaphoreType.DMA,
        ],
    )
    def sc_fill(
        stripes_hbm,
        ones_hbm,
        ri_hbm,
        ci_hbm,
        out_hbm,
        tile_v,
        stripe_v,
        ri_v,
        ci_v,
        sem,
    ):
        wid = lax.axis_index("s") * NC + lax.axis_index("c")
        pltpu.sync_copy(ones_hbm, tile_v)
        pltpu.sync_copy(ri_hbm, ri_v)
        pltpu.sync_copy(ci_hbm, ci_v)
        lane = lax.broadcasted_iota(jnp.int32, (16,), 0)
        for bb in range(BPW):
            b = wid * BPW + bb
            pltpu.sync_copy(stripes_hbm.at[b], stripe_v.at[pl.ds(0, NSTRIPE)])

            def _chunk(k, carry):
                rk = ri_v[pl.ds(k * 16, 16)]
                ck = ci_v[pl.ds(k * 16, 16)]
                v = stripe_v[pl.ds(k * 16, 16)]
                plsc.store_scatter(tile_v, [rk, ck], v)
                return carry

            lax.fori_loop(0, NCHUNK - 1, _chunk, 0)
            k0 = (NCHUNK - 1) * 16
            rk = ri_v[pl.ds(k0, 16)]
            ck = ci_v[pl.ds(k0, 16)]
            v = stripe_v[pl.ds(k0, 16)]
            plsc.store_scatter(
                tile_v, [rk, ck], v, mask=lane < (NSTRIPE - k0)
            )
            copies = [
                pltpu.async_copy(tile_v, out_hbm.at[b, l], sem)
                for l in range(NLAYERS)
            ]
            for cp in copies:
                cp.wait()

    return sc_fill


def kernel(x, keys_cls, keys_image, layer):
    B = x.shape[0]
    kc = jnp.take(keys_cls, layer, axis=0)
    ki = jnp.take(keys_image, layer, axis=0)
    kp = (
        jnp.zeros((KPAD, DIM), jnp.float32)
        .at[:POOL]
        .set(kc)
        .at[POOL : 2 * POOL]
        .set(ki)
    )

    stripes, dacc = _tc_select(x, kp)
    stripes_flat = stripes.reshape(B, NSTRIPE)

    j = np.arange(NSTRIPE_PAD)
    rj = np.minimum(j // STRIPE, NTOK - 1)
    cj = 1 + (j % STRIPE)
    ri = jnp.asarray(rj, jnp.int32)
    ci = jnp.asarray(cj, jnp.int32)
    ones = jnp.ones((NTOK, COLS), jnp.float32)

    mask = _make_sc_fill(B)(stripes_flat, ones, ri, ci)

    dist = (1.0 - dacc[0, 0] / (B * TOPK)) + (
        1.0 - dacc[0, 1] / (B * (NTOK - 1) * TOPK)
    )
    return (mask, dist)


# speedup vs baseline: 3.9994x; 2.8913x over previous
import jax
import jax.numpy as jnp
from jax.experimental import pallas as pl

POOL = 20
TOPK = 5
NTOK = 197
DIM = 768
NLAYERS = 12
COLS = NTOK + 2 * POOL
KPAD = 128


BB = 2


def _body(x_ref, kp_ref, mask_ref, dacc_ref):
    b = pl.program_id(0)

    @pl.when(b == 0)
    def _():
        dacc_ref[...] = jnp.zeros((8, 128), jnp.float32)

    for i in range(BB):
        _one_batch(x_ref, kp_ref, mask_ref, dacc_ref, i)


def _one_batch(x_ref, kp_ref, mask_ref, dacc_ref, i):
    xb = x_ref[i]
    xn = xb / jnp.maximum(jnp.sqrt(jnp.sum(xb * xb, axis=1, keepdims=True)), 1e-12)
    kp = kp_ref[...]
    kn = kp / jnp.maximum(jnp.sqrt(jnp.sum(kp * kp, axis=1, keepdims=True)), 1e-12)
    sim = jax.lax.dot_general(
        xn, kn, (((1,), (1,)), ((), ())), preferred_element_type=jnp.float32
    )

    r = jax.lax.broadcasted_iota(jnp.int32, (NTOK, KPAD), 0)
    c = jax.lax.broadcasted_iota(jnp.int32, (NTOK, KPAD), 1)
    valid = ((r == 0) & (c >= 1) & (c < 1 + POOL)) | (
        (r != 0) & (c >= 1 + POOL) & (c < 1 + 2 * POOL)
    )
    simv = jnp.where(valid, sim, -2.0)

    ind = jnp.zeros((NTOK, KPAD), jnp.float32)
    ssum = jnp.zeros((NTOK, 1), jnp.float32)
    for _ in range(TOPK):
        m = jnp.max(simv, axis=1, keepdims=True)
        first = jnp.min(jnp.where(simv == m, c, KPAD), axis=1, keepdims=True)
        onehot = c == first
        ind = jnp.where(onehot, 1.0, ind)
        ssum = ssum + m
        simv = jnp.where(onehot, -3.0, simv)

    mask128 = jnp.where((c >= 1) & (c < 1 + 2 * POOL), ind, 1.0)
    tile = jnp.concatenate(
        [mask128, jnp.full((NTOK, COLS - KPAD), 1.0, jnp.float32)], axis=1
    )
    mask_ref[i] = jnp.broadcast_to(tile[None], (NLAYERS, NTOK, COLS))

    rr = jax.lax.broadcasted_iota(jnp.int32, (NTOK, 1), 0)
    cls_sum = jnp.sum(jnp.where(rr == 0, ssum, 0.0))
    img_sum = jnp.sum(jnp.where(rr == 0, 0.0, ssum))
    ar = jax.lax.broadcasted_iota(jnp.int32, (8, 128), 0)
    ac = jax.lax.broadcasted_iota(jnp.int32, (8, 128), 1)
    part = jnp.where((ar == 0) & (ac == 0), cls_sum, 0.0) + jnp.where(
        (ar == 0) & (ac == 1), img_sum, 0.0
    )
    dacc_ref[...] += part


def kernel(x, keys_cls, keys_image, layer):
    B = x.shape[0]
    kc = jnp.take(keys_cls, layer, axis=0)
    ki = jnp.take(keys_image, layer, axis=0)
    kp = (
        jnp.zeros((KPAD, DIM), jnp.float32)
        .at[1 : 1 + POOL]
        .set(kc)
        .at[1 + POOL : 1 + 2 * POOL]
        .set(ki)
    )

    mask, dacc = pl.pallas_call(
        _body,
        grid=(B // BB,),
        in_specs=[
            pl.BlockSpec((BB, NTOK, DIM), lambda b: (b, 0, 0)),
            pl.BlockSpec((KPAD, DIM), lambda b: (0, 0)),
        ],
        out_specs=[
            pl.BlockSpec((BB, NLAYERS, NTOK, COLS), lambda b: (b, 0, 0, 0)),
            pl.BlockSpec((8, 128), lambda b: (0, 0)),
        ],
        out_shape=[
            jax.ShapeDtypeStruct((B, NLAYERS, NTOK, COLS), jnp.float32),
            jax.ShapeDtypeStruct((8, 128), jnp.float32),
        ],
    )(x, kp)

    dist = (1.0 - dacc[0, 0] / (B * TOPK)) + (
        1.0 - dacc[0, 1] / (B * (NTOK - 1) * TOPK)
    )
    return (mask, dist)


# speedup vs baseline: 4.1786x; 1.0448x over previous
import jax
import jax.numpy as jnp
from jax.experimental import pallas as pl

POOL = 20
TOPK = 5
NTOK = 197
DIM = 768
NLAYERS = 12
COLS = NTOK + 2 * POOL
KPAD = 128


BB = 4


def _body(x_ref, kp_ref, mask_ref, dacc_ref):
    b = pl.program_id(0)

    @pl.when(b == 0)
    def _():
        dacc_ref[...] = jnp.zeros((8, 128), jnp.float32)

    for i in range(BB):
        _one_batch(x_ref, kp_ref, mask_ref, dacc_ref, i)


def _one_batch(x_ref, kp_ref, mask_ref, dacc_ref, i):
    xb = x_ref[i]
    xn = xb / jnp.maximum(jnp.sqrt(jnp.sum(xb * xb, axis=1, keepdims=True)), 1e-12)
    kp = kp_ref[...]
    kn = kp / jnp.maximum(jnp.sqrt(jnp.sum(kp * kp, axis=1, keepdims=True)), 1e-12)
    sim = jax.lax.dot_general(
        xn, kn, (((1,), (1,)), ((), ())), preferred_element_type=jnp.float32
    )

    r = jax.lax.broadcasted_iota(jnp.int32, (NTOK, KPAD), 0)
    c = jax.lax.broadcasted_iota(jnp.int32, (NTOK, KPAD), 1)
    valid = ((r == 0) & (c >= 1) & (c < 1 + POOL)) | (
        (r != 0) & (c >= 1 + POOL) & (c < 1 + 2 * POOL)
    )
    simv = jnp.where(valid, sim, -2.0)

    ind = jnp.zeros((NTOK, KPAD), jnp.float32)
    ssum = jnp.zeros((NTOK, 1), jnp.float32)
    for _ in range(TOPK):
        m = jnp.max(simv, axis=1, keepdims=True)
        first = jnp.min(jnp.where(simv == m, c, KPAD), axis=1, keepdims=True)
        onehot = c == first
        ind = jnp.where(onehot, 1.0, ind)
        ssum = ssum + m
        simv = jnp.where(onehot, -3.0, simv)

    mask128 = jnp.where((c >= 1) & (c < 1 + 2 * POOL), ind, 1.0)
    tile = jnp.concatenate(
        [mask128, jnp.full((NTOK, COLS - KPAD), 1.0, jnp.float32)], axis=1
    )
    mask_ref[i] = jnp.broadcast_to(tile[None], (NLAYERS, NTOK, COLS))

    rr = jax.lax.broadcasted_iota(jnp.int32, (NTOK, 1), 0)
    cls_sum = jnp.sum(jnp.where(rr == 0, ssum, 0.0))
    img_sum = jnp.sum(jnp.where(rr == 0, 0.0, ssum))
    ar = jax.lax.broadcasted_iota(jnp.int32, (8, 128), 0)
    ac = jax.lax.broadcasted_iota(jnp.int32, (8, 128), 1)
    part = jnp.where((ar == 0) & (ac == 0), cls_sum, 0.0) + jnp.where(
        (ar == 0) & (ac == 1), img_sum, 0.0
    )
    dacc_ref[...] += part


def kernel(x, keys_cls, keys_image, layer):
    B = x.shape[0]
    kc = jnp.take(keys_cls, layer, axis=0)
    ki = jnp.take(keys_image, layer, axis=0)
    kp = (
        jnp.zeros((KPAD, DIM), jnp.float32)
        .at[1 : 1 + POOL]
        .set(kc)
        .at[1 + POOL : 1 + 2 * POOL]
        .set(ki)
    )

    mask, dacc = pl.pallas_call(
        _body,
        grid=(B // BB,),
        in_specs=[
            pl.BlockSpec((BB, NTOK, DIM), lambda b: (b, 0, 0)),
            pl.BlockSpec((KPAD, DIM), lambda b: (0, 0)),
        ],
        out_specs=[
            pl.BlockSpec((BB, NLAYERS, NTOK, COLS), lambda b: (b, 0, 0, 0)),
            pl.BlockSpec((8, 128), lambda b: (0, 0)),
        ],
        out_shape=[
            jax.ShapeDtypeStruct((B, NLAYERS, NTOK, COLS), jnp.float32),
            jax.ShapeDtypeStruct((8, 128), jnp.float32),
        ],
    )(x, kp)

    dist = (1.0 - dacc[0, 0] / (B * TOPK)) + (
        1.0 - dacc[0, 1] / (B * (NTOK - 1) * TOPK)
    )
    return (mask, dist)
